# initial kernel scaffold (unmeasured)
import jax
import jax.numpy as jnp
from jax import lax
from jax.experimental import pallas as pl
from jax.experimental.pallas import tpu as pltpu


def kernel(
    x,
):
    def body(*refs):
        pass

    out_shape = jax.ShapeDtypeStruct(..., jnp.float32)
    return pl.pallas_call(body, out_shape=out_shape)(...)


# baseline (device time: 66312 ns/iter reference)
import jax
import jax.numpy as jnp
from jax import lax
from jax.experimental import pallas as pl
from jax.experimental.pallas import tpu as pltpu

N_CHUNKS = 8


def kernel(x):
    M, N = x.shape
    H = M // 2
    Hc = H // N_CHUNKS

    def body(x_ref, out_ref, xsend, xrecv, xs_sems, xr_sems, ys_sems, yr_sems):
        mx = lax.axis_index("x")
        my = lax.axis_index("y")
        x_peer = (1 - mx, my)
        y_peer = (mx, 1 - my)

        barrier = pltpu.get_barrier_semaphore()
        for nbr in (x_peer, y_peer):
            pl.semaphore_signal(barrier, inc=1, device_id=nbr,
                                device_id_type=pl.DeviceIdType.MESH)
        pl.semaphore_wait(barrier, 2)

        base = my * H
        xsend[...] = x_ref[pl.ds(base, H), :].astype(jnp.bfloat16)

        p1 = []
        for c in range(N_CHUNKS):
            r = pl.ds(c * Hc, Hc)
            rdma = pltpu.make_async_remote_copy(
                src_ref=xsend.at[r, :],
                dst_ref=xrecv.at[r, :],
                send_sem=xs_sems.at[c],
                recv_sem=xr_sems.at[c],
                device_id=x_peer,
                device_id_type=pl.DeviceIdType.MESH,
            )
            rdma.start()
            p1.append(rdma)

        p2 = []
        for c in range(N_CHUNKS):
            r = pl.ds(c * Hc, Hc)
            p1[c].wait_recv()
            rows = pl.ds(base + c * Hc, Hc)
            out_ref[rows, :] = xsend[r, :] + xrecv[r, :]
            rdma = pltpu.make_async_remote_copy(
                src_ref=out_ref.at[rows, :],
                dst_ref=out_ref.at[rows, :],
                send_sem=ys_sems.at[c],
                recv_sem=yr_sems.at[c],
                device_id=y_peer,
                device_id_type=pl.DeviceIdType.MESH,
            )
            rdma.start()
            p2.append(rdma)

        for c in range(N_CHUNKS):
            p1[c].wait_send()
            p2[c].wait_send()
            p2[c].wait_recv()

    out_shape = jax.ShapeDtypeStruct((M, N), jnp.bfloat16)
    return pl.pallas_call(
        body,
        out_shape=out_shape,
        in_specs=[pl.BlockSpec(memory_space=pltpu.VMEM)],
        out_specs=pl.BlockSpec(memory_space=pltpu.VMEM),
        scratch_shapes=[
            pltpu.VMEM((H, N), jnp.bfloat16),
            pltpu.VMEM((H, N), jnp.bfloat16),
            pltpu.SemaphoreType.DMA((N_CHUNKS,)),
            pltpu.SemaphoreType.DMA((N_CHUNKS,)),
            pltpu.SemaphoreType.DMA((N_CHUNKS,)),
            pltpu.SemaphoreType.DMA((N_CHUNKS,)),
        ],
        compiler_params=pltpu.CompilerParams(collective_id=0),
    )(x)


# device time: 58917 ns/iter; 1.1255x vs baseline; 1.1255x over previous
import jax
import jax.numpy as jnp
from jax import lax
from jax.experimental import pallas as pl
from jax.experimental.pallas import tpu as pltpu

N_CHUNKS = 16


def kernel(x):
    M, N = x.shape
    H = M // 2
    Hc = H // N_CHUNKS

    def body(x_ref, out_ref, stage, xsend, xrecv,
             l_sems, xs_sems, xr_sems, ys_sems, yr_sems):
        mx = lax.axis_index("x")
        my = lax.axis_index("y")
        x_peer = (1 - mx, my)
        y_peer = (mx, 1 - my)

        barrier = pltpu.get_barrier_semaphore()
        for nbr in (x_peer, y_peer):
            pl.semaphore_signal(barrier, inc=1, device_id=nbr,
                                device_id_type=pl.DeviceIdType.MESH)
        pl.semaphore_wait(barrier, 2)

        base = my * H

        loads = []
        for c in range(N_CHUNKS):
            r = pl.ds(c * Hc, Hc)
            cp = pltpu.make_async_copy(
                x_ref.at[pl.ds(base + c * Hc, Hc), :],
                stage.at[r, :],
                l_sems.at[c],
            )
            cp.start()
            loads.append(cp)

        p1 = []
        for c in range(N_CHUNKS):
            r = pl.ds(c * Hc, Hc)
            loads[c].wait()
            xsend[r, :] = stage[r, :].astype(jnp.bfloat16)
            rdma = pltpu.make_async_remote_copy(
                src_ref=xsend.at[r, :],
                dst_ref=xrecv.at[r, :],
                send_sem=xs_sems.at[c],
                recv_sem=xr_sems.at[c],
                device_id=x_peer,
                device_id_type=pl.DeviceIdType.MESH,
            )
            rdma.start()
            p1.append(rdma)

        p2 = []
        for c in range(N_CHUNKS):
            r = pl.ds(c * Hc, Hc)
            p1[c].wait_recv()
            rows = pl.ds(base + c * Hc, Hc)
            out_ref[rows, :] = xsend[r, :] + xrecv[r, :]
            rdma = pltpu.make_async_remote_copy(
                src_ref=out_ref.at[rows, :],
                dst_ref=out_ref.at[rows, :],
                send_sem=ys_sems.at[c],
                recv_sem=yr_sems.at[c],
                device_id=y_peer,
                device_id_type=pl.DeviceIdType.MESH,
            )
            rdma.start()
            p2.append(rdma)

        for c in range(N_CHUNKS):
            p1[c].wait_send()
            p2[c].wait_send()
            p2[c].wait_recv()

    out_shape = jax.ShapeDtypeStruct((M, N), jnp.bfloat16)
    return pl.pallas_call(
        body,
        out_shape=out_shape,
        in_specs=[pl.BlockSpec(memory_space=pl.ANY)],
        out_specs=pl.BlockSpec(memory_space=pltpu.VMEM),
        scratch_shapes=[
            pltpu.VMEM((H, N), jnp.float32),
            pltpu.VMEM((H, N), jnp.bfloat16),
            pltpu.VMEM((H, N), jnp.bfloat16),
            pltpu.SemaphoreType.DMA((N_CHUNKS,)),
            pltpu.SemaphoreType.DMA((N_CHUNKS,)),
            pltpu.SemaphoreType.DMA((N_CHUNKS,)),
            pltpu.SemaphoreType.DMA((N_CHUNKS,)),
            pltpu.SemaphoreType.DMA((N_CHUNKS,)),
        ],
        compiler_params=pltpu.CompilerParams(collective_id=0),
    )(x)


# device time: 58372 ns/iter; 1.1360x vs baseline; 1.0093x over previous
import jax
import jax.numpy as jnp
from jax import lax
from jax.experimental import pallas as pl
from jax.experimental.pallas import tpu as pltpu

N_CHUNKS = 32


def kernel(x):
    M, N = x.shape
    H = M // 2
    Hc = H // N_CHUNKS

    def body(x_ref, out_ref, stage, xsend, xrecv, ssum,
             l_sems, o_sems, xs_sems, xr_sems, ys_sems, yr_sems):
        mx = lax.axis_index("x")
        my = lax.axis_index("y")
        x_peer = (1 - mx, my)
        y_peer = (mx, 1 - my)

        barrier = pltpu.get_barrier_semaphore()
        for nbr in (x_peer, y_peer):
            pl.semaphore_signal(barrier, inc=1, device_id=nbr,
                                device_id_type=pl.DeviceIdType.MESH)
        pl.semaphore_wait(barrier, 2)

        base = my * H

        loads = []
        for c in range(N_CHUNKS):
            r = pl.ds(c * Hc, Hc)
            cp = pltpu.make_async_copy(
                x_ref.at[pl.ds(base + c * Hc, Hc), :],
                stage.at[r, :],
                l_sems.at[c],
            )
            cp.start()
            loads.append(cp)

        p1 = []
        for c in range(N_CHUNKS):
            r = pl.ds(c * Hc, Hc)
            loads[c].wait()
            xsend[r, :] = stage[r, :].astype(jnp.bfloat16)
            rdma = pltpu.make_async_remote_copy(
                src_ref=xsend.at[r, :],
                dst_ref=xrecv.at[r, :],
                send_sem=xs_sems.at[c],
                recv_sem=xr_sems.at[c],
                device_id=x_peer,
                device_id_type=pl.DeviceIdType.MESH,
            )
            rdma.start()
            p1.append(rdma)

        p2 = []
        ostores = []
        for c in range(N_CHUNKS):
            r = pl.ds(c * Hc, Hc)
            p1[c].wait_recv()
            rows = pl.ds(base + c * Hc, Hc)
            ssum[r, :] = xsend[r, :] + xrecv[r, :]
            rdma = pltpu.make_async_remote_copy(
                src_ref=ssum.at[r, :],
                dst_ref=out_ref.at[rows, :],
                send_sem=ys_sems.at[c],
                recv_sem=yr_sems.at[c],
                device_id=y_peer,
                device_id_type=pl.DeviceIdType.MESH,
            )
            rdma.start()
            p2.append(rdma)
            ocp = pltpu.make_async_copy(
                ssum.at[r, :], out_ref.at[rows, :], o_sems.at[c],
            )
            ocp.start()
            ostores.append(ocp)

        for c in range(N_CHUNKS):
            p1[c].wait_send()
            p2[c].wait_send()
            p2[c].wait_recv()
            ostores[c].wait()

    out_shape = jax.ShapeDtypeStruct((M, N), jnp.bfloat16)
    return pl.pallas_call(
        body,
        out_shape=out_shape,
        in_specs=[pl.BlockSpec(memory_space=pl.ANY)],
        out_specs=pl.BlockSpec(memory_space=pl.ANY),
        scratch_shapes=[
            pltpu.VMEM((H, N), jnp.float32),
            pltpu.VMEM((H, N), jnp.bfloat16),
            pltpu.VMEM((H, N), jnp.bfloat16),
            pltpu.VMEM((H, N), jnp.bfloat16),
            pltpu.SemaphoreType.DMA((N_CHUNKS,)),
            pltpu.SemaphoreType.DMA((N_CHUNKS,)),
            pltpu.SemaphoreType.DMA((N_CHUNKS,)),
            pltpu.SemaphoreType.DMA((N_CHUNKS,)),
            pltpu.SemaphoreType.DMA((N_CHUNKS,)),
            pltpu.SemaphoreType.DMA((N_CHUNKS,)),
        ],
        compiler_params=pltpu.CompilerParams(collective_id=0),
    )(x)
